# baseline (speedup 1.0000x reference)
import jax
import jax.numpy as jnp
from jax import lax
from jax.experimental import pallas as pl
from jax.experimental.pallas import tpu as pltpu
from jax.experimental.pallas import tpu_sc as plsc

N = 10000
E = 320000
D = 128
C = 4
OUT = D * C

NC = 2
NS = 16
L = 16

EDGES_PER_TILE = E // NS
BLK = 80
NBLK = EDGES_PER_TILE // BLK
NZCH = N // BLK
PASSES = C // NC


def _mm_body(x_ref, w_ref, o_ref):
    o_ref[...] = jnp.dot(x_ref[...], w_ref[...],
                         preferred_element_type=jnp.float32)


def _project(x, w):
    return pl.pallas_call(
        _mm_body,
        grid=(10,),
        in_specs=[
            pl.BlockSpec((N // 10, D), lambda i: (i, 0)),
            pl.BlockSpec((D, D), lambda i: (0, 0)),
        ],
        out_specs=pl.BlockSpec((N // 10, D), lambda i: (i, 0)),
        out_shape=jax.ShapeDtypeStruct((N, D), jnp.float32),
    )(x, w)


def _sc_scatter(h, src, dst, attr_flat):
    mesh = plsc.VectorSubcoreMesh(core_axis_name="c", subcore_axis_name="s",
                                  num_cores=NC, num_subcores=NS)

    def body(h_hbm, src_hbm, dst_hbm, attr_hbm, out_hbm,
             acc, srcb0, srcb1, srcb2, dstb0, dstb1, dstb2,
             dsts0, dsts1, dsts2, attrb0, attrb1, attrb2,
             rows0, rows1, rows2,
             isem0, isem1, isem2, gsem0, gsem1, gsem2,
             ssem0, ssem1, ssem2):
        core = lax.axis_index("c")
        s = lax.axis_index("s")
        zeros16 = jnp.zeros((L,), jnp.float32)
        srcb = [srcb0, srcb1, srcb2]
        dstb = [dstb0, dstb1, dstb2]
        dsts = [dsts0, dsts1, dsts2]
        attrb = [attrb0, attrb1, attrb2]
        rows = [rows0, rows1, rows2]
        isem = [isem0, isem1, isem2]
        gsem = [gsem0, gsem1, gsem2]
        ssem = [ssem0, ssem1, ssem2]
        Q = 3

        def idx_base(b):
            return s * EDGES_PER_TILE + b * BLK

        def fire_idx(slot, b):
            base = idx_base(b)
            pltpu.async_copy(src_hbm.at[pl.ds(base, BLK)], srcb[slot],
                             isem[slot])
            pltpu.async_copy(dst_hbm.at[pl.ds(base, BLK)], dstb[slot],
                             isem[slot])
            pltpu.async_copy(attr_hbm.at[pl.ds(base * C, BLK * C)],
                             attrb[slot], isem[slot])

        def wait_idx(slot, b):
            base = idx_base(b)
            pltpu.make_async_copy(src_hbm.at[pl.ds(base, BLK)], srcb[slot],
                                  isem[slot]).wait()
            pltpu.make_async_copy(dst_hbm.at[pl.ds(base, BLK)], dstb[slot],
                                  isem[slot]).wait()
            pltpu.make_async_copy(attr_hbm.at[pl.ds(base * C, BLK * C)],
                                  attrb[slot], isem[slot]).wait()

        def fire_gather(slot):
            pltpu.async_copy(h_hbm.at[srcb[slot]], rows[slot], gsem[slot])

        def wait_gather(slot):
            pltpu.make_async_copy(h_hbm.at[srcb[slot]], rows[slot],
                                  gsem[slot]).wait()

        def fire_scatter(slot):
            pltpu.async_copy(rows[slot], acc.at[dsts[slot]], ssem[slot],
                             add=True)

        def wait_scatter(slot):
            pltpu.make_async_copy(rows[slot], acc.at[dsts[slot]],
                                  ssem[slot]).wait()

        for p in range(PASSES):
            ch = PASSES * core + p
            chv = jnp.full((L,), 1, dtype=jnp.int32) * ch

            def zrow(r, _):
                for j in range(D // L):
                    rows0[r, pl.ds(L * j, L)] = zeros16
                return 0
            lax.fori_loop(0, BLK, zrow, 0)

            def zcp(i, carry):
                k = s + NS * i

                @pl.when(k < NZCH)
                def _():
                    off = pl.multiple_of(k * BLK, 8)
                    pltpu.sync_copy(rows0, acc.at[pl.ds(off, BLK), :])
                return carry
            lax.fori_loop(0, (NZCH + NS - 1) // NS, zcp, 0)
            plsc.subcore_barrier()

            def compute_scale(slot):
                def edge_body(q, carry):
                    scales = []
                    for u in range(16):
                        e = 16 * q + u
                        ev = jnp.full((L,), e * C, dtype=jnp.int32)
                        scales.append(
                            plsc.load_gather(attrb[slot], [ev + chv]))
                    for u in range(16):
                        e = 16 * q + u
                        for j in range(D // L):
                            rows[slot][e, pl.ds(L * j, L)] = (
                                rows[slot][e, pl.ds(L * j, L)] * scales[u])
                    return carry
                lax.fori_loop(0, BLK // 16, edge_body, 0)

            def snap_dst(slot):
                for g in range(BLK // L):
                    dsts[slot][pl.ds(L * g, L)] = dstb[slot][pl.ds(L * g, L)]

            def halfstep(t, u):
                b = Q * t + u
                nxt = (u + 1) % Q
                wait_gather(u)
                compute_scale(u)
                snap_dst(u)
                fire_scatter(u)

                @pl.when(b + Q < NBLK)
                def _():
                    fire_idx(u, b + Q)

                @pl.when(b + 1 < NBLK)
                def _():
                    @pl.when(b >= 2)
                    def _():
                        wait_scatter(nxt)
                    wait_idx(nxt, b + 1)
                    fire_gather(nxt)

            fire_idx(0, 0)
            fire_idx(1, 1)
            fire_idx(2, 2)
            wait_idx(0, 0)
            fire_gather(0)

            def trip_body(t, carry):
                for u in range(Q):
                    halfstep(t, u)
                return carry

            lax.fori_loop(0, NBLK // Q, trip_body, 0)
            halfstep(NBLK // Q, 0)
            wait_scatter(1)
            wait_scatter(2)
            wait_scatter(0)
            plsc.subcore_barrier()

            def wcp(i, carry):
                k = s + NS * i

                @pl.when(k < NZCH)
                def _():
                    off = pl.multiple_of(k * BLK, 8)
                    dof = pl.multiple_of(ch * N + k * BLK, 8)
                    pltpu.sync_copy(acc.at[pl.ds(off, BLK), :],
                                    out_hbm.at[pl.ds(dof, BLK), :])
                return carry
            lax.fori_loop(0, (NZCH + NS - 1) // NS, wcp, 0)
            if p + 1 < PASSES:
                plsc.subcore_barrier()

    f = pl.kernel(
        body,
        out_type=jax.ShapeDtypeStruct((C * N, D), jnp.float32),
        mesh=mesh,
        compiler_params=pltpu.CompilerParams(needs_layout_passes=False),
        scratch_types=dict(
            acc=pltpu.VMEM_SHARED((N, D), jnp.float32),
            srcb0=pltpu.VMEM((BLK,), jnp.int32),
            srcb1=pltpu.VMEM((BLK,), jnp.int32),
            srcb2=pltpu.VMEM((BLK,), jnp.int32),
            dstb0=pltpu.VMEM((BLK,), jnp.int32),
            dstb1=pltpu.VMEM((BLK,), jnp.int32),
            dstb2=pltpu.VMEM((BLK,), jnp.int32),
            dsts0=pltpu.VMEM((BLK,), jnp.int32),
            dsts1=pltpu.VMEM((BLK,), jnp.int32),
            dsts2=pltpu.VMEM((BLK,), jnp.int32),
            attrb0=pltpu.VMEM((BLK * C,), jnp.float32),
            attrb1=pltpu.VMEM((BLK * C,), jnp.float32),
            attrb2=pltpu.VMEM((BLK * C,), jnp.float32),
            rows0=pltpu.VMEM((BLK, D), jnp.float32),
            rows1=pltpu.VMEM((BLK, D), jnp.float32),
            rows2=pltpu.VMEM((BLK, D), jnp.float32),
            isem0=pltpu.SemaphoreType.DMA,
            isem1=pltpu.SemaphoreType.DMA,
            isem2=pltpu.SemaphoreType.DMA,
            gsem0=pltpu.SemaphoreType.DMA,
            gsem1=pltpu.SemaphoreType.DMA,
            gsem2=pltpu.SemaphoreType.DMA,
            ssem0=pltpu.SemaphoreType.DMA,
            ssem1=pltpu.SemaphoreType.DMA,
            ssem2=pltpu.SemaphoreType.DMA,
        ),
    )
    return f(h, src, dst, attr_flat)


def _fin_body(a0, a1, a2, a3, b_ref, o_ref):
    av = [a0, a1, a2, a3]
    for c in range(C):
        o_ref[:, c * D:(c + 1) * D] = jnp.maximum(
            av[c][...] + b_ref[:, c * D:(c + 1) * D], 0.0)


def _finish(acc, b2d):
    blk = N // 10
    in_specs = (
        [pl.BlockSpec((blk, D), lambda i, c=c: (i + c * 10, 0))
         for c in range(C)]
        + [pl.BlockSpec((1, OUT), lambda i: (0, 0))]
    )
    return pl.pallas_call(
        _fin_body,
        grid=(10,),
        in_specs=in_specs,
        out_specs=pl.BlockSpec((blk, OUT), lambda i: (i, 0)),
        out_shape=jax.ShapeDtypeStruct((N, OUT), jnp.float32),
    )(acc, acc, acc, acc, b2d)


def kernel(node_features, edge_index, edge_attr, W, b):
    h = _project(node_features, W)
    src = edge_index[0].astype(jnp.int32)
    dst = edge_index[1].astype(jnp.int32)
    attr_flat = edge_attr.reshape(-1)
    acc = _sc_scatter(h, src, dst, attr_flat)
    return _finish(acc, b.reshape(1, OUT))
